# baseline (speedup 1.0000x reference)
import functools

import jax
import jax.numpy as jnp
from jax import lax
from jax.experimental import pallas as pl
from jax.experimental.pallas import tpu as pltpu
from jax.experimental.pallas import tpu_sc as plsc

BATCH = 16384
EMB_K = 64
PAIRW = 2 * EMB_K
NUM_CORES = 2
NUM_SUBCORES = 16
NW = NUM_CORES * NUM_SUBCORES
BPW = BATCH // NW
WLEN = 144
CHUNK = 256
NCH = BPW // CHUNK


def _run(uidx_hbm, iidx_hbm, ut_hbm, it_hbm, w_hbm, out_hbm,
         uidx_v, iidx_v, upidx_v, ipidx_v, rows_v, w_v, out_v, sem):
    wid = lax.axis_index("s") * NUM_CORES + lax.axis_index("c")
    base = wid * BPW
    pltpu.sync_copy(uidx_hbm.at[pl.ds(base, BPW)], uidx_v)
    pltpu.sync_copy(iidx_hbm.at[pl.ds(base, BPW)], iidx_v)

    def halve(j, carry):
        s = pl.ds(j * 16, 16)
        upidx_v[s] = jax.lax.shift_right_logical(uidx_v[s], 1)
        ipidx_v[s] = jax.lax.shift_right_logical(iidx_v[s], 1)
        return carry

    lax.fori_loop(0, BPW // 16, halve, 0)
    pltpu.sync_copy(w_hbm, w_v)

    wvecs = [w_v[pl.ds(j * 16, 16)] for j in range(WLEN // 16)]
    wu = [wvecs[k // 16][k % 16] for k in range(EMB_K)]
    wi = [wvecs[(EMB_K + k) // 16][k % 16] for k in range(EMB_K)]
    bias = wvecs[(2 * EMB_K) // 16][0]
    lane = lax.iota(jnp.int32, 16)

    for h in range(NCH):
        cbase = h * CHUNK
        cu = pltpu.async_copy(
            ut_hbm.at[upidx_v.at[pl.ds(cbase, CHUNK)]], rows_v.at[0], sem)
        ci = pltpu.async_copy(
            it_hbm.at[ipidx_v.at[pl.ds(cbase, CHUNK)]], rows_v.at[1], sem)
        cu.wait()
        ci.wait()

        def body(g, carry):
            s = pl.ds(cbase + g * 16, 16)
            rows = lane + g * 16
            uoff = (uidx_v[s] & 1) * EMB_K
            ioff = (iidx_v[s] & 1) * EMB_K
            acc = jnp.full((16,), 0.0, jnp.float32) + bias
            for k in range(EMB_K):
                ucol = plsc.load_gather(rows_v.at[0], [rows, uoff + k])
                icol = plsc.load_gather(rows_v.at[1], [rows, ioff + k])
                acc = acc + ucol * wu[k] + icol * wi[k]
            out_v[s] = 1.0 / (1.0 + jnp.exp(-acc))
            return carry

        lax.fori_loop(0, CHUNK // 16, body, 0)

    pltpu.sync_copy(out_v, out_hbm.at[pl.ds(base, BPW)])


@jax.jit
def _launch(u_idx, i_idx, user_table, item_table, wb):
    mesh = plsc.VectorSubcoreMesh(
        core_axis_name="c", subcore_axis_name="s",
        num_cores=NUM_CORES, num_subcores=NUM_SUBCORES)
    kern = functools.partial(
        pl.kernel,
        out_type=jax.ShapeDtypeStruct((BATCH,), jnp.float32),
        mesh=mesh,
        compiler_params=pltpu.CompilerParams(needs_layout_passes=False),
        scratch_types=[
            pltpu.VMEM((BPW,), jnp.int32),
            pltpu.VMEM((BPW,), jnp.int32),
            pltpu.VMEM((BPW,), jnp.int32),
            pltpu.VMEM((BPW,), jnp.int32),
            pltpu.VMEM((2, CHUNK, PAIRW), jnp.float32),
            pltpu.VMEM((WLEN,), jnp.float32),
            pltpu.VMEM((BPW,), jnp.float32),
            pltpu.SemaphoreType.DMA,
        ],
    )(_run)
    return kern(u_idx, i_idx, user_table, item_table, wb)


def kernel(x, user_table, item_table, W, b):
    u_idx = x[:, 0].astype(jnp.int32)
    i_idx = x[:, 1].astype(jnp.int32)
    ut2 = user_table.reshape(user_table.shape[0] // 2, PAIRW) + 0.0
    it2 = item_table.reshape(item_table.shape[0] // 2, PAIRW) + 0.0
    wb = jnp.concatenate(
        [W.reshape(-1), b.reshape(-1),
         jnp.zeros((WLEN - 2 * EMB_K - 1,), jnp.float32)])
    return _launch(u_idx, i_idx, ut2, it2, wb)


# speedup vs baseline: 2.6530x; 2.6530x over previous
import functools

import jax
import jax.numpy as jnp
from jax import lax
from jax.experimental import pallas as pl
from jax.experimental.pallas import tpu as pltpu
from jax.experimental.pallas import tpu_sc as plsc

BATCH = 16384
EMB_K = 64
NROWS = 1000000
NUM_CORES = 2
NUM_SUBCORES = 16
NW = NUM_CORES * NUM_SUBCORES
BPW = BATCH // NW
WLEN = 144
CW = 512
SEG = 31232
NCH = SEG // CW
CAP = 768
PAD = 16
BIGI = 1 << 30


def _scan16k(idx_v, lo, hi, ml_idx, ml_pos, lane):
    def body(j, cnt):
        s = pl.ds(j * 16, 16)
        iv = idx_v[s]
        m = (iv >= lo) & (iv < hi)
        w = pl.ds(cnt, 16)
        plsc.store_compressed(ml_idx.at[w], iv, mask=m)
        plsc.store_compressed(ml_pos.at[w], lane + j * 16, mask=m)
        return cnt + plsc.all_reduce_population_count(m)[0]
    return lax.fori_loop(0, BATCH // 16, body, 0)


def _k1(uidx_hbm, iidx_hbm, utT_hbm, itT_hbm, w_hbm,
        upos_hbm, uval_hbm, ipos_hbm, ival_hbm,
        idx_v, stage_v, ml_idx, ml_pos, cl_col, cl_pos,
        opos_v, oval_v, w_v, sem):
    wid = lax.axis_index("s") * NUM_CORES + lax.axis_index("c")
    seg_lo = wid * SEG
    nch = jnp.where(wid == NW - 1, NCH + 2, NCH)
    seg_hi = jnp.where(wid == NW - 1, NROWS, seg_lo + SEG)
    pltpu.sync_copy(w_hbm, w_v)
    wvecs = [w_v[pl.ds(j * 16, 16)] for j in range(WLEN // 16)]
    lane = lax.iota(jnp.int32, 16)
    zero16 = jnp.full((16,), 0, jnp.int32)
    zf16 = jnp.full((16,), 0.0, jnp.float32)

    def do_pass(idx_hbm, tT_hbm, wofs, pos_hbm, val_hbm):
        ws = [wvecs[(wofs + k) // 16][(wofs + k) % 16] for k in range(EMB_K)]
        pltpu.sync_copy(idx_hbm, idx_v)

        def init(j, carry):
            s = pl.ds(j * 16, 16)
            opos_v[s] = zero16
            oval_v[s] = zf16
            return carry
        lax.fori_loop(0, (CAP + PAD) // 16, init, 0)

        cnt = _scan16k(idx_v, seg_lo, seg_hi, ml_idx, ml_pos, lane)

        def fill(j, carry):
            s = pl.ds(j * 16, 16)
            m = lane + j * 16 >= cnt
            iv = ml_idx[s]
            ml_idx[s] = jnp.where(m, BIGI, iv)
            return carry
        lax.fori_loop(0, (CAP + PAD) // 16, fill, 0)

        def chunk(ch, ocnt):
            lo = seg_lo + ch * CW

            @pl.when(ch < NCH + 1)
            def _():
                pltpu.sync_copy(tT_hbm.at[:, pl.ds(lo, CW)], stage_v)

            @pl.when(ch >= NCH + 1)
            def _():
                pltpu.sync_copy(tT_hbm.at[:, pl.ds(lo, 128)],
                                stage_v.at[:, pl.ds(0, 128)])

            def cscan(t, ccnt):
                s = pl.ds(t * 16, 16)
                iv = ml_idx[s]
                m = (iv >= lo) & (iv < lo + CW)
                w = pl.ds(ccnt, 16)
                plsc.store_compressed(cl_col.at[w], iv - lo, mask=m)
                plsc.store_compressed(cl_pos.at[w], ml_pos[s], mask=m)
                return ccnt + plsc.all_reduce_population_count(m)[0]
            ccnt = lax.fori_loop(0, (CAP + PAD) // 16, cscan, 0)

            def grp(g, ocnt_in):
                rem = ccnt - g * 16
                lm = lane < rem
                cols = jnp.where(lm, cl_col[pl.ds(g * 16, 16)], 0)
                pos = jnp.where(lm, cl_pos[pl.ds(g * 16, 16)], 0)
                acc = zf16
                for k in range(EMB_K):
                    col = plsc.load_gather(stage_v, [jnp.full((16,), k,
                                                             jnp.int32), cols])
                    acc = acc + col * ws[k]
                acc = jnp.where(lm, acc, 0.0)
                w = pl.ds(ocnt_in, 16)
                opos_v[w] = pos
                oval_v[w] = acc
                return ocnt_in + jnp.minimum(rem, 16)
            ngrp = (ccnt + 15) >> 4
            return lax.fori_loop(0, ngrp, grp, ocnt)

        lax.fori_loop(0, nch, chunk, 0)
        pltpu.sync_copy(opos_v.at[pl.ds(0, CAP)],
                        pos_hbm.at[pl.ds(wid * CAP, CAP)])
        pltpu.sync_copy(oval_v.at[pl.ds(0, CAP)],
                        val_hbm.at[pl.ds(wid * CAP, CAP)])

    do_pass(uidx_hbm, utT_hbm, 0, upos_hbm, uval_hbm)
    do_pass(iidx_hbm, itT_hbm, EMB_K, ipos_hbm, ival_hbm)


def _k2(upos_hbm, uval_hbm, ipos_hbm, ival_hbm, w_hbm, out_hbm,
        pos_v, val_v, w_v, acc_v, out_v):
    wid = lax.axis_index("s") * NUM_CORES + lax.axis_index("c")
    base = wid * BPW
    pltpu.sync_copy(w_hbm, w_v)
    bias = w_v[pl.ds(2 * EMB_K - 8, 16)][8]
    lane = lax.iota(jnp.int32, 16)

    def init(g, carry):
        acc_v[pl.ds(g * 16, 16)] = jnp.full((16,), 0.0, jnp.float32) + bias
        return carry
    lax.fori_loop(0, BPW // 16, init, 0)

    for pos_hbm, val_hbm in ((upos_hbm, uval_hbm), (ipos_hbm, ival_hbm)):
        pltpu.sync_copy(pos_hbm, pos_v)
        pltpu.sync_copy(val_hbm, val_v)

        def merge(t, carry):
            s = pl.ds(t * 16, 16)
            pos = pos_v[s]
            m = (pos >= base) & (pos < base + BPW)
            plsc.addupdate_scatter(acc_v, [pos - base], val_v[s], mask=m)
            return carry
        lax.fori_loop(0, NW * CAP // 16, merge, 0)

    def finish(g, carry):
        s = pl.ds(g * 16, 16)
        out_v[s] = 1.0 / (1.0 + jnp.exp(-acc_v[s]))
        return carry
    lax.fori_loop(0, BPW // 16, finish, 0)
    pltpu.sync_copy(out_v, out_hbm.at[pl.ds(base, BPW)])


@jax.jit
def _launch(u_idx, i_idx, utT, itT, wb):
    mesh = plsc.VectorSubcoreMesh(
        core_axis_name="c", subcore_axis_name="s",
        num_cores=NUM_CORES, num_subcores=NUM_SUBCORES)
    cparams = pltpu.CompilerParams(needs_layout_passes=False)
    lists = jax.ShapeDtypeStruct((NW * CAP,), jnp.int32)
    listsf = jax.ShapeDtypeStruct((NW * CAP,), jnp.float32)
    k1 = functools.partial(
        pl.kernel,
        out_type=(lists, listsf, lists, listsf),
        mesh=mesh,
        compiler_params=cparams,
        scratch_types=[
            pltpu.VMEM((BATCH,), jnp.int32),
            pltpu.VMEM((EMB_K, CW), jnp.float32),
            pltpu.VMEM((CAP + PAD,), jnp.int32),
            pltpu.VMEM((CAP + PAD,), jnp.int32),
            pltpu.VMEM((CAP + PAD,), jnp.int32),
            pltpu.VMEM((CAP + PAD,), jnp.int32),
            pltpu.VMEM((CAP + PAD,), jnp.int32),
            pltpu.VMEM((CAP + PAD,), jnp.float32),
            pltpu.VMEM((WLEN,), jnp.float32),
            pltpu.SemaphoreType.DMA,
        ],
    )(_k1)
    upos, uval, ipos, ival = k1(u_idx, i_idx, utT, itT, wb)
    k2 = functools.partial(
        pl.kernel,
        out_type=jax.ShapeDtypeStruct((BATCH,), jnp.float32),
        mesh=mesh,
        compiler_params=cparams,
        scratch_types=[
            pltpu.VMEM((NW * CAP,), jnp.int32),
            pltpu.VMEM((NW * CAP,), jnp.float32),
            pltpu.VMEM((WLEN,), jnp.float32),
            pltpu.VMEM((BPW,), jnp.float32),
            pltpu.VMEM((BPW,), jnp.float32),
        ],
    )(_k2)
    return k2(upos, uval, ipos, ival, wb)


def kernel(x, user_table, item_table, W, b):
    u_idx = x[:, 0].astype(jnp.int32)
    i_idx = x[:, 1].astype(jnp.int32)
    wb = jnp.concatenate(
        [W.reshape(-1), b.reshape(-1),
         jnp.zeros((WLEN - 2 * EMB_K - 1,), jnp.float32)])
    return _launch(u_idx, i_idx, user_table.T, item_table.T, wb)


# speedup vs baseline: 2.9320x; 1.1051x over previous
import functools

import jax
import jax.numpy as jnp
from jax import lax
from jax.experimental import pallas as pl
from jax.experimental.pallas import tpu as pltpu
from jax.experimental.pallas import tpu_sc as plsc

BATCH = 16384
EMB_K = 64
NROWS = 1000000
NUM_CORES = 2
NUM_SUBCORES = 16
NW = NUM_CORES * NUM_SUBCORES
BPW = BATCH // NW
WLEN = 144
CW = 512
SEG = 31232
NCH = SEG // CW
CAP = 768
PAD = 16
BIGI = 1 << 30


def _scan16k(idx_v, lo, hi, ml_idx, ml_pos, lane):
    def body(j, cnt):
        s = pl.ds(j * 16, 16)
        iv = idx_v[s]
        m = (iv >= lo) & (iv < hi)
        w = pl.ds(cnt, 16)
        plsc.store_compressed(ml_idx.at[w], iv, mask=m)
        plsc.store_compressed(ml_pos.at[w], lane + j * 16, mask=m)
        return cnt + plsc.all_reduce_population_count(m)[0]
    return lax.fori_loop(0, BATCH // 16, body, 0)


def _k1(uidx_hbm, iidx_hbm, utT_hbm, itT_hbm, w_hbm,
        upos_hbm, uval_hbm, ipos_hbm, ival_hbm,
        idx_v, ustage_v, istage_v, uml_idx, uml_pos, iml_idx, iml_pos,
        cl_col, cl_pos, uopos_v, uoval_v, iopos_v, ioval_v, w_v, sem):
    wid = lax.axis_index("s") * NUM_CORES + lax.axis_index("c")
    seg_lo = wid * SEG
    nch = jnp.where(wid == NW - 1, NCH + 2, NCH)
    seg_hi = jnp.where(wid == NW - 1, NROWS, seg_lo + SEG)
    pltpu.sync_copy(w_hbm, w_v)
    wvecs = [w_v[pl.ds(j * 16, 16)] for j in range(WLEN // 16)]
    lane = lax.iota(jnp.int32, 16)
    zero16 = jnp.full((16,), 0, jnp.int32)
    zf16 = jnp.full((16,), 0.0, jnp.float32)

    wu = [wvecs[k // 16][k % 16] for k in range(EMB_K)]
    wi = [wvecs[(EMB_K + k) // 16][(EMB_K + k) % 16] for k in range(EMB_K)]

    def init(j, carry):
        s = pl.ds(j * 16, 16)
        uopos_v[s] = zero16
        uoval_v[s] = zf16
        iopos_v[s] = zero16
        ioval_v[s] = zf16
        return carry
    lax.fori_loop(0, (CAP + PAD) // 16, init, 0)

    pltpu.sync_copy(uidx_hbm, idx_v)
    ucnt = _scan16k(idx_v, seg_lo, seg_hi, uml_idx, uml_pos, lane)
    pltpu.sync_copy(iidx_hbm, idx_v)
    icnt = _scan16k(idx_v, seg_lo, seg_hi, iml_idx, iml_pos, lane)

    def fill(j, carry):
        s = pl.ds(j * 16, 16)
        mu = lane + j * 16 >= ucnt
        mi = lane + j * 16 >= icnt
        uml_idx[s] = jnp.where(mu, BIGI, uml_idx[s])
        iml_idx[s] = jnp.where(mi, BIGI, iml_idx[s])
        return carry
    lax.fori_loop(0, (CAP + PAD) // 16, fill, 0)

    def half_chunk(lo, stage, ml_idx, ml_pos, ws, opos_v, oval_v, ocnt):
        def cscan(t, ccnt):
            s = pl.ds(t * 16, 16)
            iv = ml_idx[s]
            m = (iv >= lo) & (iv < lo + CW)
            w = pl.ds(ccnt, 16)
            plsc.store_compressed(cl_col.at[w], iv - lo, mask=m)
            plsc.store_compressed(cl_pos.at[w], ml_pos[s], mask=m)
            return ccnt + plsc.all_reduce_population_count(m)[0]
        ccnt = lax.fori_loop(0, (CAP + PAD) // 16, cscan, 0)

        def grp(g, ocnt_in):
            rem = ccnt - g * 16
            lm = lane < rem
            cols = jnp.where(lm, cl_col[pl.ds(g * 16, 16)], 0)
            pos = jnp.where(lm, cl_pos[pl.ds(g * 16, 16)], 0)
            acc = zf16
            for k in range(EMB_K):
                col = plsc.load_gather(stage, [jnp.full((16,), k,
                                                        jnp.int32), cols])
                acc = acc + col * ws[k]
            acc = jnp.where(lm, acc, 0.0)
            w = pl.ds(ocnt_in, 16)
            opos_v[w] = pos
            oval_v[w] = acc
            return ocnt_in + jnp.minimum(rem, 16)
        ngrp = (ccnt + 15) >> 4
        return lax.fori_loop(0, ngrp, grp, ocnt)

    def chunk(ch, carry):
        uocnt, iocnt = carry
        lo = seg_lo + ch * CW

        @pl.when(ch < NCH + 1)
        def _():
            cu = pltpu.async_copy(utT_hbm.at[:, pl.ds(lo, CW)], ustage_v, sem)
            ci = pltpu.async_copy(itT_hbm.at[:, pl.ds(lo, CW)], istage_v, sem)
            cu.wait()
            ci.wait()

        @pl.when(ch >= NCH + 1)
        def _():
            cu = pltpu.async_copy(utT_hbm.at[:, pl.ds(lo, 128)],
                                  ustage_v.at[:, pl.ds(0, 128)], sem)
            ci = pltpu.async_copy(itT_hbm.at[:, pl.ds(lo, 128)],
                                  istage_v.at[:, pl.ds(0, 128)], sem)
            cu.wait()
            ci.wait()

        uocnt = half_chunk(lo, ustage_v, uml_idx, uml_pos, wu,
                           uopos_v, uoval_v, uocnt)
        iocnt = half_chunk(lo, istage_v, iml_idx, iml_pos, wi,
                           iopos_v, ioval_v, iocnt)
        return (uocnt, iocnt)

    lax.fori_loop(0, nch, chunk, (0, 0))
    pltpu.sync_copy(uopos_v.at[pl.ds(0, CAP)],
                    upos_hbm.at[pl.ds(wid * CAP, CAP)])
    pltpu.sync_copy(uoval_v.at[pl.ds(0, CAP)],
                    uval_hbm.at[pl.ds(wid * CAP, CAP)])
    pltpu.sync_copy(iopos_v.at[pl.ds(0, CAP)],
                    ipos_hbm.at[pl.ds(wid * CAP, CAP)])
    pltpu.sync_copy(ioval_v.at[pl.ds(0, CAP)],
                    ival_hbm.at[pl.ds(wid * CAP, CAP)])


def _k2(upos_hbm, uval_hbm, ipos_hbm, ival_hbm, w_hbm, out_hbm,
        pos_v, val_v, w_v, acc_v, out_v):
    wid = lax.axis_index("s") * NUM_CORES + lax.axis_index("c")
    base = wid * BPW
    pltpu.sync_copy(w_hbm, w_v)
    bias = w_v[pl.ds(2 * EMB_K - 8, 16)][8]
    lane = lax.iota(jnp.int32, 16)

    def init(g, carry):
        acc_v[pl.ds(g * 16, 16)] = jnp.full((16,), 0.0, jnp.float32) + bias
        return carry
    lax.fori_loop(0, BPW // 16, init, 0)

    for pos_hbm, val_hbm in ((upos_hbm, uval_hbm), (ipos_hbm, ival_hbm)):
        pltpu.sync_copy(pos_hbm, pos_v)
        pltpu.sync_copy(val_hbm, val_v)

        def merge(t, carry):
            s = pl.ds(t * 16, 16)
            pos = pos_v[s]
            m = (pos >= base) & (pos < base + BPW)
            plsc.addupdate_scatter(acc_v, [pos - base], val_v[s], mask=m)
            return carry
        lax.fori_loop(0, NW * CAP // 16, merge, 0)

    def finish(g, carry):
        s = pl.ds(g * 16, 16)
        out_v[s] = 1.0 / (1.0 + jnp.exp(-acc_v[s]))
        return carry
    lax.fori_loop(0, BPW // 16, finish, 0)
    pltpu.sync_copy(out_v, out_hbm.at[pl.ds(base, BPW)])


@jax.jit
def _launch(u_idx, i_idx, utT, itT, wb):
    mesh = plsc.VectorSubcoreMesh(
        core_axis_name="c", subcore_axis_name="s",
        num_cores=NUM_CORES, num_subcores=NUM_SUBCORES)
    cparams = pltpu.CompilerParams(needs_layout_passes=False)
    lists = jax.ShapeDtypeStruct((NW * CAP,), jnp.int32)
    listsf = jax.ShapeDtypeStruct((NW * CAP,), jnp.float32)
    k1 = functools.partial(
        pl.kernel,
        out_type=(lists, listsf, lists, listsf),
        mesh=mesh,
        compiler_params=cparams,
        scratch_types=[
            pltpu.VMEM((BATCH,), jnp.int32),
            pltpu.VMEM((EMB_K, CW), jnp.float32),
            pltpu.VMEM((EMB_K, CW), jnp.float32),
            pltpu.VMEM((CAP + PAD,), jnp.int32),
            pltpu.VMEM((CAP + PAD,), jnp.int32),
            pltpu.VMEM((CAP + PAD,), jnp.int32),
            pltpu.VMEM((CAP + PAD,), jnp.int32),
            pltpu.VMEM((CAP + PAD,), jnp.int32),
            pltpu.VMEM((CAP + PAD,), jnp.int32),
            pltpu.VMEM((CAP + PAD,), jnp.int32),
            pltpu.VMEM((CAP + PAD,), jnp.float32),
            pltpu.VMEM((CAP + PAD,), jnp.int32),
            pltpu.VMEM((CAP + PAD,), jnp.float32),
            pltpu.VMEM((WLEN,), jnp.float32),
            pltpu.SemaphoreType.DMA,
        ],
    )(_k1)
    upos, uval, ipos, ival = k1(u_idx, i_idx, utT, itT, wb)
    k2 = functools.partial(
        pl.kernel,
        out_type=jax.ShapeDtypeStruct((BATCH,), jnp.float32),
        mesh=mesh,
        compiler_params=cparams,
        scratch_types=[
            pltpu.VMEM((NW * CAP,), jnp.int32),
            pltpu.VMEM((NW * CAP,), jnp.float32),
            pltpu.VMEM((WLEN,), jnp.float32),
            pltpu.VMEM((BPW,), jnp.float32),
            pltpu.VMEM((BPW,), jnp.float32),
        ],
    )(_k2)
    return k2(upos, uval, ipos, ival, wb)


def kernel(x, user_table, item_table, W, b):
    u_idx = x[:, 0].astype(jnp.int32)
    i_idx = x[:, 1].astype(jnp.int32)
    wb = jnp.concatenate(
        [W.reshape(-1), b.reshape(-1),
         jnp.zeros((WLEN - 2 * EMB_K - 1,), jnp.float32)])
    return _launch(u_idx, i_idx, user_table.T, item_table.T, wb)


# speedup vs baseline: 3.3461x; 1.1412x over previous
import functools

import jax
import jax.numpy as jnp
from jax import lax
from jax.experimental import pallas as pl
from jax.experimental.pallas import tpu as pltpu
from jax.experimental.pallas import tpu_sc as plsc

BATCH = 16384
EMB_K = 64
NROWS = 1000000
NUM_CORES = 2
NUM_SUBCORES = 16
NW = NUM_CORES * NUM_SUBCORES
BPW = BATCH // NW
WLEN = 144
CW = 256
SEG = 31232
NCH = SEG // CW
CAP = 768
PAD = 16
BIGI = 1 << 30


def _scan16k(idx_v, lo, hi, ml_idx, ml_pos, lane):
    def body(j, cnt):
        s = pl.ds(j * 16, 16)
        iv = idx_v[s]
        m = (iv >= lo) & (iv < hi)
        w = pl.ds(cnt, 16)
        plsc.store_compressed(ml_idx.at[w], iv, mask=m)
        plsc.store_compressed(ml_pos.at[w], lane + j * 16, mask=m)
        return cnt + plsc.all_reduce_population_count(m)[0]
    return lax.fori_loop(0, BATCH // 16, body, 0)


def _k1(uidx_hbm, iidx_hbm, utT_hbm, itT_hbm, w_hbm,
        upos_hbm, uval_hbm, ipos_hbm, ival_hbm,
        idx_v, ustage_v, istage_v, uml_idx, uml_pos, iml_idx, iml_pos,
        cl_col, cl_pos, uopos_v, uoval_v, iopos_v, ioval_v, w_v, sem, sem2):
    wid = lax.axis_index("s") * NUM_CORES + lax.axis_index("c")
    seg_lo = wid * SEG
    nch = jnp.where(wid == NW - 1, NCH + 3, NCH)
    seg_hi = jnp.where(wid == NW - 1, NROWS, seg_lo + SEG)
    pltpu.sync_copy(w_hbm, w_v)
    wvecs = [w_v[pl.ds(j * 16, 16)] for j in range(WLEN // 16)]
    lane = lax.iota(jnp.int32, 16)
    zero16 = jnp.full((16,), 0, jnp.int32)
    zf16 = jnp.full((16,), 0.0, jnp.float32)

    wu = [wvecs[k // 16][k % 16] for k in range(EMB_K)]
    wi = [wvecs[(EMB_K + k) // 16][(EMB_K + k) % 16] for k in range(EMB_K)]

    def init(j, carry):
        s = pl.ds(j * 16, 16)
        uopos_v[s] = zero16
        uoval_v[s] = zf16
        iopos_v[s] = zero16
        ioval_v[s] = zf16
        return carry
    lax.fori_loop(0, (CAP + PAD) // 16, init, 0)

    pltpu.sync_copy(uidx_hbm, idx_v)
    ucnt = _scan16k(idx_v, seg_lo, seg_hi, uml_idx, uml_pos, lane)
    pltpu.sync_copy(iidx_hbm, idx_v)
    icnt = _scan16k(idx_v, seg_lo, seg_hi, iml_idx, iml_pos, lane)

    def fill(j, carry):
        s = pl.ds(j * 16, 16)
        mu = lane + j * 16 >= ucnt
        mi = lane + j * 16 >= icnt
        uml_idx[s] = jnp.where(mu, BIGI, uml_idx[s])
        iml_idx[s] = jnp.where(mi, BIGI, iml_idx[s])
        return carry
    lax.fori_loop(0, (CAP + PAD) // 16, fill, 0)

    def half_chunk(lo, stage, ml_idx, ml_pos, ws, opos_v, oval_v, ocnt):
        def cscan(t, ccnt):
            s = pl.ds(t * 16, 16)
            iv = ml_idx[s]
            m = (iv >= lo) & (iv < lo + CW)
            w = pl.ds(ccnt, 16)
            plsc.store_compressed(cl_col.at[w], iv - lo, mask=m)
            plsc.store_compressed(cl_pos.at[w], ml_pos[s], mask=m)
            return ccnt + plsc.all_reduce_population_count(m)[0]
        ccnt = lax.fori_loop(0, (CAP + PAD) // 16, cscan, 0)

        def grp(g, ocnt_in):
            rem = ccnt - g * 16
            lm = lane < rem
            cols = jnp.where(lm, cl_col[pl.ds(g * 16, 16)], 0)
            pos = jnp.where(lm, cl_pos[pl.ds(g * 16, 16)], 0)
            acc = zf16
            for k in range(EMB_K):
                col = plsc.load_gather(stage, [jnp.full((16,), k,
                                                        jnp.int32), cols])
                acc = acc + col * ws[k]
            acc = jnp.where(lm, acc, 0.0)
            w = pl.ds(ocnt_in, 16)
            opos_v[w] = pos
            oval_v[w] = acc
            return ocnt_in + jnp.minimum(rem, 16)
        ngrp = (ccnt + 15) >> 4
        return lax.fori_loop(0, ngrp, grp, ocnt)

    def issue(c, su, si, sm):
        lo = seg_lo + c * CW

        @pl.when(c < jnp.minimum(nch, NCH + 2))
        def _():
            pltpu.async_copy(utT_hbm.at[:, pl.ds(lo, CW)], su, sm)
            pltpu.async_copy(itT_hbm.at[:, pl.ds(lo, CW)], si, sm)

        @pl.when((c >= NCH + 2) & (c < nch))
        def _():
            pltpu.async_copy(utT_hbm.at[:, pl.ds(lo, 128)],
                             su.at[:, pl.ds(0, 128)], sm)
            pltpu.async_copy(itT_hbm.at[:, pl.ds(lo, 128)],
                             si.at[:, pl.ds(0, 128)], sm)

    def drain(c, su, si, sm):
        @pl.when(c < jnp.minimum(nch, NCH + 2))
        def _():
            pltpu.make_async_copy(utT_hbm.at[:, pl.ds(0, CW)], su, sm).wait()
            pltpu.make_async_copy(utT_hbm.at[:, pl.ds(0, CW)], si, sm).wait()

        @pl.when((c >= NCH + 2) & (c < nch))
        def _():
            pltpu.make_async_copy(utT_hbm.at[:, pl.ds(0, 128)],
                                  su.at[:, pl.ds(0, 128)], sm).wait()
            pltpu.make_async_copy(utT_hbm.at[:, pl.ds(0, 128)],
                                  si.at[:, pl.ds(0, 128)], sm).wait()

    def do_chunk(c, uocnt, iocnt, su, si):
        lo = seg_lo + c * CW
        uocnt = half_chunk(lo, su, uml_idx, uml_pos, wu,
                           uopos_v, uoval_v, uocnt)
        iocnt = half_chunk(lo, si, iml_idx, iml_pos, wi,
                           iopos_v, ioval_v, iocnt)
        return uocnt, iocnt

    issue(0, ustage_v.at[0], istage_v.at[0], sem)

    def pair(j, carry):
        uocnt, iocnt = carry
        c0 = 2 * j
        c1 = c0 + 1
        issue(c1, ustage_v.at[1], istage_v.at[1], sem2)
        drain(c0, ustage_v.at[0], istage_v.at[0], sem)
        uocnt, iocnt = do_chunk(c0, uocnt, iocnt,
                                ustage_v.at[0], istage_v.at[0])
        issue(c0 + 2, ustage_v.at[0], istage_v.at[0], sem)
        drain(c1, ustage_v.at[1], istage_v.at[1], sem2)
        uocnt, iocnt = do_chunk(c1, uocnt, iocnt,
                                ustage_v.at[1], istage_v.at[1])
        return (uocnt, iocnt)

    lax.fori_loop(0, (nch + 1) >> 1, pair, (0, 0))
    pltpu.sync_copy(uopos_v.at[pl.ds(0, CAP)],
                    upos_hbm.at[pl.ds(wid * CAP, CAP)])
    pltpu.sync_copy(uoval_v.at[pl.ds(0, CAP)],
                    uval_hbm.at[pl.ds(wid * CAP, CAP)])
    pltpu.sync_copy(iopos_v.at[pl.ds(0, CAP)],
                    ipos_hbm.at[pl.ds(wid * CAP, CAP)])
    pltpu.sync_copy(ioval_v.at[pl.ds(0, CAP)],
                    ival_hbm.at[pl.ds(wid * CAP, CAP)])


def _k2(upos_hbm, uval_hbm, ipos_hbm, ival_hbm, w_hbm, out_hbm,
        pos_v, val_v, w_v, acc_v, out_v):
    wid = lax.axis_index("s") * NUM_CORES + lax.axis_index("c")
    base = wid * BPW
    pltpu.sync_copy(w_hbm, w_v)
    bias = w_v[pl.ds(2 * EMB_K - 8, 16)][8]
    lane = lax.iota(jnp.int32, 16)

    def init(g, carry):
        acc_v[pl.ds(g * 16, 16)] = jnp.full((16,), 0.0, jnp.float32) + bias
        return carry
    lax.fori_loop(0, BPW // 16, init, 0)

    for pos_hbm, val_hbm in ((upos_hbm, uval_hbm), (ipos_hbm, ival_hbm)):
        pltpu.sync_copy(pos_hbm, pos_v)
        pltpu.sync_copy(val_hbm, val_v)

        def merge(t, carry):
            s = pl.ds(t * 16, 16)
            pos = pos_v[s]
            m = (pos >= base) & (pos < base + BPW)
            plsc.addupdate_scatter(acc_v, [pos - base], val_v[s], mask=m)
            return carry
        lax.fori_loop(0, NW * CAP // 16, merge, 0)

    def finish(g, carry):
        s = pl.ds(g * 16, 16)
        out_v[s] = 1.0 / (1.0 + jnp.exp(-acc_v[s]))
        return carry
    lax.fori_loop(0, BPW // 16, finish, 0)
    pltpu.sync_copy(out_v, out_hbm.at[pl.ds(base, BPW)])


@jax.jit
def _launch(u_idx, i_idx, utT, itT, wb):
    mesh = plsc.VectorSubcoreMesh(
        core_axis_name="c", subcore_axis_name="s",
        num_cores=NUM_CORES, num_subcores=NUM_SUBCORES)
    cparams = pltpu.CompilerParams(needs_layout_passes=False)
    lists = jax.ShapeDtypeStruct((NW * CAP,), jnp.int32)
    listsf = jax.ShapeDtypeStruct((NW * CAP,), jnp.float32)
    k1 = functools.partial(
        pl.kernel,
        out_type=(lists, listsf, lists, listsf),
        mesh=mesh,
        compiler_params=cparams,
        scratch_types=[
            pltpu.VMEM((BATCH,), jnp.int32),
            pltpu.VMEM((2, EMB_K, CW), jnp.float32),
            pltpu.VMEM((2, EMB_K, CW), jnp.float32),
            pltpu.VMEM((CAP + PAD,), jnp.int32),
            pltpu.VMEM((CAP + PAD,), jnp.int32),
            pltpu.VMEM((CAP + PAD,), jnp.int32),
            pltpu.VMEM((CAP + PAD,), jnp.int32),
            pltpu.VMEM((CAP + PAD,), jnp.int32),
            pltpu.VMEM((CAP + PAD,), jnp.int32),
            pltpu.VMEM((CAP + PAD,), jnp.int32),
            pltpu.VMEM((CAP + PAD,), jnp.float32),
            pltpu.VMEM((CAP + PAD,), jnp.int32),
            pltpu.VMEM((CAP + PAD,), jnp.float32),
            pltpu.VMEM((WLEN,), jnp.float32),
            pltpu.SemaphoreType.DMA,
            pltpu.SemaphoreType.DMA,
        ],
    )(_k1)
    upos, uval, ipos, ival = k1(u_idx, i_idx, utT, itT, wb)
    k2 = functools.partial(
        pl.kernel,
        out_type=jax.ShapeDtypeStruct((BATCH,), jnp.float32),
        mesh=mesh,
        compiler_params=cparams,
        scratch_types=[
            pltpu.VMEM((NW * CAP,), jnp.int32),
            pltpu.VMEM((NW * CAP,), jnp.float32),
            pltpu.VMEM((WLEN,), jnp.float32),
            pltpu.VMEM((BPW,), jnp.float32),
            pltpu.VMEM((BPW,), jnp.float32),
        ],
    )(_k2)
    return k2(upos, uval, ipos, ival, wb)


def kernel(x, user_table, item_table, W, b):
    u_idx = x[:, 0].astype(jnp.int32)
    i_idx = x[:, 1].astype(jnp.int32)
    wb = jnp.concatenate(
        [W.reshape(-1), b.reshape(-1),
         jnp.zeros((WLEN - 2 * EMB_K - 1,), jnp.float32)])
    return _launch(u_idx, i_idx, user_table.T, item_table.T, wb)


# speedup vs baseline: 3.5070x; 1.0481x over previous
import functools

import jax
import jax.numpy as jnp
from jax import lax
from jax.experimental import pallas as pl
from jax.experimental.pallas import tpu as pltpu
from jax.experimental.pallas import tpu_sc as plsc

BATCH = 16384
EMB_K = 64
NROWS = 1000000
NUM_CORES = 2
NUM_SUBCORES = 16
NW = NUM_CORES * NUM_SUBCORES
BPW = BATCH // NW
WLEN = 144
CW = 256
SEG = 31232
NCH = SEG // CW
CAP = 768
PAD = 16
BIGI = 1 << 30


def _scan16k(idx_v, lo, hi, ml_idx, ml_pos, lane):
    def body(j, cnt):
        s = pl.ds(j * 16, 16)
        iv = idx_v[s]
        m = (iv >= lo) & (iv < hi)
        w = pl.ds(cnt, 16)
        plsc.store_compressed(ml_idx.at[w], iv, mask=m)
        plsc.store_compressed(ml_pos.at[w], lane + j * 16, mask=m)
        return cnt + plsc.all_reduce_population_count(m)[0]
    return lax.fori_loop(0, BATCH // 16, body, 0)


def _k1(uidx_hbm, iidx_hbm, utT_hbm, itT_hbm, w_hbm,
        upos_hbm, uval_hbm, ipos_hbm, ival_hbm,
        idx_v, ustage_v, istage_v, uml_idx, uml_pos, iml_idx, iml_pos,
        cl_col, cl_pos, uopos_v, uoval_v, iopos_v, ioval_v, w_v, sem, sem2):
    wid = lax.axis_index("s") * NUM_CORES + lax.axis_index("c")
    seg_lo = wid * SEG
    nch = jnp.where(wid == NW - 1, NCH + 3, NCH)
    seg_hi = jnp.where(wid == NW - 1, NROWS, seg_lo + SEG)
    pltpu.sync_copy(w_hbm, w_v)
    wvecs = [w_v[pl.ds(j * 16, 16)] for j in range(WLEN // 16)]
    lane = lax.iota(jnp.int32, 16)
    zero16 = jnp.full((16,), 0, jnp.int32)
    zf16 = jnp.full((16,), 0.0, jnp.float32)

    wu = [wvecs[k // 16][k % 16] for k in range(EMB_K)]
    wi = [wvecs[(EMB_K + k) // 16][(EMB_K + k) % 16] for k in range(EMB_K)]

    def init(j, carry):
        s = pl.ds(j * 16, 16)
        uopos_v[s] = zero16
        uoval_v[s] = zf16
        iopos_v[s] = zero16
        ioval_v[s] = zf16
        return carry
    lax.fori_loop(0, (CAP + PAD) // 16, init, 0)

    pltpu.sync_copy(uidx_hbm, idx_v)
    ucnt = _scan16k(idx_v, seg_lo, seg_hi, uml_idx, uml_pos, lane)
    pltpu.sync_copy(iidx_hbm, idx_v)
    icnt = _scan16k(idx_v, seg_lo, seg_hi, iml_idx, iml_pos, lane)

    def fill(j, carry):
        s = pl.ds(j * 16, 16)
        mu = lane + j * 16 >= ucnt
        mi = lane + j * 16 >= icnt
        uml_idx[s] = jnp.where(mu, BIGI, uml_idx[s])
        iml_idx[s] = jnp.where(mi, BIGI, iml_idx[s])
        return carry
    lax.fori_loop(0, (CAP + PAD) // 16, fill, 0)

    def half_chunk(lo, stage, ml_idx, ml_pos, nscan, ws, opos_v, oval_v,
                   ocnt):
        def cscan(t, ccnt):
            s = pl.ds(t * 16, 16)
            iv = ml_idx[s]
            m = (iv >= lo) & (iv < lo + CW)
            w = pl.ds(ccnt, 16)
            plsc.store_compressed(cl_col.at[w], iv - lo, mask=m)
            plsc.store_compressed(cl_pos.at[w], ml_pos[s], mask=m)
            return ccnt + plsc.all_reduce_population_count(m)[0]
        ccnt = lax.fori_loop(0, nscan, cscan, 0)

        def grp(g, ocnt_in):
            rem = ccnt - g * 16
            lm = lane < rem
            cols = jnp.where(lm, cl_col[pl.ds(g * 16, 16)], 0)
            pos = jnp.where(lm, cl_pos[pl.ds(g * 16, 16)], 0)
            acc = zf16
            for k in range(EMB_K):
                col = plsc.load_gather(stage, [jnp.full((16,), k,
                                                        jnp.int32), cols])
                acc = acc + col * ws[k]
            acc = jnp.where(lm, acc, 0.0)
            w = pl.ds(ocnt_in, 16)
            opos_v[w] = pos
            oval_v[w] = acc
            return ocnt_in + jnp.minimum(rem, 16)
        ngrp = (ccnt + 15) >> 4
        return lax.fori_loop(0, ngrp, grp, ocnt)

    def issue(c, su, si, sm):
        lo = seg_lo + c * CW

        @pl.when(c < jnp.minimum(nch, NCH + 2))
        def _():
            pltpu.async_copy(utT_hbm.at[:, pl.ds(lo, CW)], su, sm)
            pltpu.async_copy(itT_hbm.at[:, pl.ds(lo, CW)], si, sm)

        @pl.when((c >= NCH + 2) & (c < nch))
        def _():
            pltpu.async_copy(utT_hbm.at[:, pl.ds(lo, 128)],
                             su.at[:, pl.ds(0, 128)], sm)
            pltpu.async_copy(itT_hbm.at[:, pl.ds(lo, 128)],
                             si.at[:, pl.ds(0, 128)], sm)

    def drain(c, su, si, sm):
        @pl.when(c < jnp.minimum(nch, NCH + 2))
        def _():
            pltpu.make_async_copy(utT_hbm.at[:, pl.ds(0, CW)], su, sm).wait()
            pltpu.make_async_copy(utT_hbm.at[:, pl.ds(0, CW)], si, sm).wait()

        @pl.when((c >= NCH + 2) & (c < nch))
        def _():
            pltpu.make_async_copy(utT_hbm.at[:, pl.ds(0, 128)],
                                  su.at[:, pl.ds(0, 128)], sm).wait()
            pltpu.make_async_copy(utT_hbm.at[:, pl.ds(0, 128)],
                                  si.at[:, pl.ds(0, 128)], sm).wait()

    unscan = (ucnt + 15) >> 4
    inscan = (icnt + 15) >> 4

    def do_chunk(c, uocnt, iocnt, su, si):
        lo = seg_lo + c * CW
        uocnt = half_chunk(lo, su, uml_idx, uml_pos, unscan, wu,
                           uopos_v, uoval_v, uocnt)
        iocnt = half_chunk(lo, si, iml_idx, iml_pos, inscan, wi,
                           iopos_v, ioval_v, iocnt)
        return uocnt, iocnt

    issue(0, ustage_v.at[0], istage_v.at[0], sem)

    def pair(j, carry):
        uocnt, iocnt = carry
        c0 = 2 * j
        c1 = c0 + 1
        issue(c1, ustage_v.at[1], istage_v.at[1], sem2)
        drain(c0, ustage_v.at[0], istage_v.at[0], sem)
        uocnt, iocnt = do_chunk(c0, uocnt, iocnt,
                                ustage_v.at[0], istage_v.at[0])
        issue(c0 + 2, ustage_v.at[0], istage_v.at[0], sem)
        drain(c1, ustage_v.at[1], istage_v.at[1], sem2)
        uocnt, iocnt = do_chunk(c1, uocnt, iocnt,
                                ustage_v.at[1], istage_v.at[1])
        return (uocnt, iocnt)

    lax.fori_loop(0, (nch + 1) >> 1, pair, (0, 0))
    pltpu.sync_copy(uopos_v.at[pl.ds(0, CAP)],
                    upos_hbm.at[pl.ds(wid * CAP, CAP)])
    pltpu.sync_copy(uoval_v.at[pl.ds(0, CAP)],
                    uval_hbm.at[pl.ds(wid * CAP, CAP)])
    pltpu.sync_copy(iopos_v.at[pl.ds(0, CAP)],
                    ipos_hbm.at[pl.ds(wid * CAP, CAP)])
    pltpu.sync_copy(ioval_v.at[pl.ds(0, CAP)],
                    ival_hbm.at[pl.ds(wid * CAP, CAP)])


def _k2(upos_hbm, uval_hbm, ipos_hbm, ival_hbm, w_hbm, out_hbm,
        pos_v, val_v, w_v, acc_v, out_v):
    wid = lax.axis_index("s") * NUM_CORES + lax.axis_index("c")
    base = wid * BPW
    pltpu.sync_copy(w_hbm, w_v)
    bias = w_v[pl.ds(2 * EMB_K - 8, 16)][8]
    lane = lax.iota(jnp.int32, 16)

    def init(g, carry):
        acc_v[pl.ds(g * 16, 16)] = jnp.full((16,), 0.0, jnp.float32) + bias
        return carry
    lax.fori_loop(0, BPW // 16, init, 0)

    for pos_hbm, val_hbm in ((upos_hbm, uval_hbm), (ipos_hbm, ival_hbm)):
        pltpu.sync_copy(pos_hbm, pos_v)
        pltpu.sync_copy(val_hbm, val_v)

        def merge(t, carry):
            s = pl.ds(t * 16, 16)
            pos = pos_v[s]
            m = (pos >= base) & (pos < base + BPW)
            plsc.addupdate_scatter(acc_v, [pos - base], val_v[s], mask=m)
            return carry
        lax.fori_loop(0, NW * CAP // 16, merge, 0)

    def finish(g, carry):
        s = pl.ds(g * 16, 16)
        out_v[s] = 1.0 / (1.0 + jnp.exp(-acc_v[s]))
        return carry
    lax.fori_loop(0, BPW // 16, finish, 0)
    pltpu.sync_copy(out_v, out_hbm.at[pl.ds(base, BPW)])


@jax.jit
def _launch(u_idx, i_idx, utT, itT, wb):
    mesh = plsc.VectorSubcoreMesh(
        core_axis_name="c", subcore_axis_name="s",
        num_cores=NUM_CORES, num_subcores=NUM_SUBCORES)
    cparams = pltpu.CompilerParams(needs_layout_passes=False)
    lists = jax.ShapeDtypeStruct((NW * CAP,), jnp.int32)
    listsf = jax.ShapeDtypeStruct((NW * CAP,), jnp.float32)
    k1 = functools.partial(
        pl.kernel,
        out_type=(lists, listsf, lists, listsf),
        mesh=mesh,
        compiler_params=cparams,
        scratch_types=[
            pltpu.VMEM((BATCH,), jnp.int32),
            pltpu.VMEM((2, EMB_K, CW), jnp.float32),
            pltpu.VMEM((2, EMB_K, CW), jnp.float32),
            pltpu.VMEM((CAP + PAD,), jnp.int32),
            pltpu.VMEM((CAP + PAD,), jnp.int32),
            pltpu.VMEM((CAP + PAD,), jnp.int32),
            pltpu.VMEM((CAP + PAD,), jnp.int32),
            pltpu.VMEM((CAP + PAD,), jnp.int32),
            pltpu.VMEM((CAP + PAD,), jnp.int32),
            pltpu.VMEM((CAP + PAD,), jnp.int32),
            pltpu.VMEM((CAP + PAD,), jnp.float32),
            pltpu.VMEM((CAP + PAD,), jnp.int32),
            pltpu.VMEM((CAP + PAD,), jnp.float32),
            pltpu.VMEM((WLEN,), jnp.float32),
            pltpu.SemaphoreType.DMA,
            pltpu.SemaphoreType.DMA,
        ],
    )(_k1)
    upos, uval, ipos, ival = k1(u_idx, i_idx, utT, itT, wb)
    k2 = functools.partial(
        pl.kernel,
        out_type=jax.ShapeDtypeStruct((BATCH,), jnp.float32),
        mesh=mesh,
        compiler_params=cparams,
        scratch_types=[
            pltpu.VMEM((NW * CAP,), jnp.int32),
            pltpu.VMEM((NW * CAP,), jnp.float32),
            pltpu.VMEM((WLEN,), jnp.float32),
            pltpu.VMEM((BPW,), jnp.float32),
            pltpu.VMEM((BPW,), jnp.float32),
        ],
    )(_k2)
    return k2(upos, uval, ipos, ival, wb)


def kernel(x, user_table, item_table, W, b):
    u_idx = x[:, 0].astype(jnp.int32)
    i_idx = x[:, 1].astype(jnp.int32)
    wb = jnp.concatenate(
        [W.reshape(-1), b.reshape(-1),
         jnp.zeros((WLEN - 2 * EMB_K - 1,), jnp.float32)])
    return _launch(u_idx, i_idx, user_table.T, item_table.T, wb)


# speedup vs baseline: 3.5111x; 1.0012x over previous
import functools

import jax
import jax.numpy as jnp
from jax import lax
from jax.experimental import pallas as pl
from jax.experimental.pallas import tpu as pltpu
from jax.experimental.pallas import tpu_sc as plsc

BATCH = 16384
EMB_K = 64
NROWS = 1000000
NUM_CORES = 2
NUM_SUBCORES = 16
NW = NUM_CORES * NUM_SUBCORES
BPW = BATCH // NW
WLEN = 144
CW = 512
SEG = 31232
NCH = SEG // CW
CAP = 768
PAD = 16
BIGI = 1 << 30


def _scan16k(idx_v, lo, hi, ml_idx, ml_pos, lane):
    def body(j, cnt):
        s = pl.ds(j * 16, 16)
        iv = idx_v[s]
        m = (iv >= lo) & (iv < hi)
        w = pl.ds(cnt, 16)
        plsc.store_compressed(ml_idx.at[w], iv, mask=m)
        plsc.store_compressed(ml_pos.at[w], lane + j * 16, mask=m)
        return cnt + plsc.all_reduce_population_count(m)[0]
    return lax.fori_loop(0, BATCH // 16, body, 0)


def _k1(uidx_hbm, iidx_hbm, utT_hbm, itT_hbm, w_hbm,
        upos_hbm, uval_hbm, ipos_hbm, ival_hbm,
        idx_v, ustage_v, istage_v, uml_idx, uml_pos, iml_idx, iml_pos,
        cl_col, cl_pos, uopos_v, uoval_v, iopos_v, ioval_v, w_v, sem, sem2):
    wid = lax.axis_index("s") * NUM_CORES + lax.axis_index("c")
    seg_lo = wid * SEG
    nch = jnp.where(wid == NW - 1, NCH + 2, NCH)
    seg_hi = jnp.where(wid == NW - 1, NROWS, seg_lo + SEG)
    pltpu.sync_copy(w_hbm, w_v)
    wvecs = [w_v[pl.ds(j * 16, 16)] for j in range(WLEN // 16)]
    lane = lax.iota(jnp.int32, 16)
    zero16 = jnp.full((16,), 0, jnp.int32)
    zf16 = jnp.full((16,), 0.0, jnp.float32)

    wu = [wvecs[k // 16][k % 16] for k in range(EMB_K)]
    wi = [wvecs[(EMB_K + k) // 16][(EMB_K + k) % 16] for k in range(EMB_K)]

    def init(j, carry):
        s = pl.ds(j * 16, 16)
        uopos_v[s] = zero16
        uoval_v[s] = zf16
        iopos_v[s] = zero16
        ioval_v[s] = zf16
        return carry
    lax.fori_loop(0, (CAP + PAD) // 16, init, 0)

    pltpu.sync_copy(uidx_hbm, idx_v)
    ucnt = _scan16k(idx_v, seg_lo, seg_hi, uml_idx, uml_pos, lane)
    pltpu.sync_copy(iidx_hbm, idx_v)
    icnt = _scan16k(idx_v, seg_lo, seg_hi, iml_idx, iml_pos, lane)

    def fill(j, carry):
        s = pl.ds(j * 16, 16)
        mu = lane + j * 16 >= ucnt
        mi = lane + j * 16 >= icnt
        uml_idx[s] = jnp.where(mu, BIGI, uml_idx[s])
        iml_idx[s] = jnp.where(mi, BIGI, iml_idx[s])
        return carry
    lax.fori_loop(0, (CAP + PAD) // 16, fill, 0)

    def half_chunk(lo, stage, ml_idx, ml_pos, nscan, ws, opos_v, oval_v,
                   ocnt):
        def cscan(t, ccnt):
            s = pl.ds(t * 16, 16)
            iv = ml_idx[s]
            m = (iv >= lo) & (iv < lo + CW)
            w = pl.ds(ccnt, 16)
            plsc.store_compressed(cl_col.at[w], iv - lo, mask=m)
            plsc.store_compressed(cl_pos.at[w], ml_pos[s], mask=m)
            return ccnt + plsc.all_reduce_population_count(m)[0]
        ccnt = lax.fori_loop(0, nscan, cscan, 0)

        def grp(g, ocnt_in):
            rem = ccnt - g * 16
            lm = lane < rem
            cols = jnp.where(lm, cl_col[pl.ds(g * 16, 16)], 0)
            pos = jnp.where(lm, cl_pos[pl.ds(g * 16, 16)], 0)
            acc = zf16
            for k in range(EMB_K):
                col = plsc.load_gather(stage, [jnp.full((16,), k,
                                                        jnp.int32), cols])
                acc = acc + col * ws[k]
            acc = jnp.where(lm, acc, 0.0)
            w = pl.ds(ocnt_in, 16)
            opos_v[w] = pos
            oval_v[w] = acc
            return ocnt_in + jnp.minimum(rem, 16)
        ngrp = (ccnt + 15) >> 4
        return lax.fori_loop(0, ngrp, grp, ocnt)

    def issue(c, tab_hbm, st, sm):
        lo = seg_lo + c * CW

        @pl.when(c < jnp.minimum(nch, NCH + 1))
        def _():
            pltpu.async_copy(tab_hbm.at[:, pl.ds(lo, CW)], st, sm)

        @pl.when((c >= NCH + 1) & (c < nch))
        def _():
            pltpu.async_copy(tab_hbm.at[:, pl.ds(lo, 128)],
                             st.at[:, pl.ds(0, 128)], sm)

    def drain(c, st, sm):
        @pl.when(c < jnp.minimum(nch, NCH + 1))
        def _():
            pltpu.make_async_copy(utT_hbm.at[:, pl.ds(0, CW)], st, sm).wait()

        @pl.when((c >= NCH + 1) & (c < nch))
        def _():
            pltpu.make_async_copy(utT_hbm.at[:, pl.ds(0, 128)],
                                  st.at[:, pl.ds(0, 128)], sm).wait()

    unscan = (ucnt + 15) >> 4
    inscan = (icnt + 15) >> 4

    issue(0, utT_hbm, ustage_v, sem)

    def chunk(c, carry):
        uocnt, iocnt = carry
        lo = seg_lo + c * CW
        issue(c, itT_hbm, istage_v, sem2)
        drain(c, ustage_v, sem)
        uocnt = half_chunk(lo, ustage_v, uml_idx, uml_pos, unscan, wu,
                           uopos_v, uoval_v, uocnt)
        issue(c + 1, utT_hbm, ustage_v, sem)
        drain(c, istage_v, sem2)
        iocnt = half_chunk(lo, istage_v, iml_idx, iml_pos, inscan, wi,
                           iopos_v, ioval_v, iocnt)
        return (uocnt, iocnt)

    lax.fori_loop(0, nch, chunk, (0, 0))
    pltpu.sync_copy(uopos_v.at[pl.ds(0, CAP)],
                    upos_hbm.at[pl.ds(wid * CAP, CAP)])
    pltpu.sync_copy(uoval_v.at[pl.ds(0, CAP)],
                    uval_hbm.at[pl.ds(wid * CAP, CAP)])
    pltpu.sync_copy(iopos_v.at[pl.ds(0, CAP)],
                    ipos_hbm.at[pl.ds(wid * CAP, CAP)])
    pltpu.sync_copy(ioval_v.at[pl.ds(0, CAP)],
                    ival_hbm.at[pl.ds(wid * CAP, CAP)])


def _k2(upos_hbm, uval_hbm, ipos_hbm, ival_hbm, w_hbm, out_hbm,
        pos_v, val_v, w_v, acc_v, out_v):
    wid = lax.axis_index("s") * NUM_CORES + lax.axis_index("c")
    base = wid * BPW
    pltpu.sync_copy(w_hbm, w_v)
    bias = w_v[pl.ds(2 * EMB_K - 8, 16)][8]
    lane = lax.iota(jnp.int32, 16)

    def init(g, carry):
        acc_v[pl.ds(g * 16, 16)] = jnp.full((16,), 0.0, jnp.float32) + bias
        return carry
    lax.fori_loop(0, BPW // 16, init, 0)

    for pos_hbm, val_hbm in ((upos_hbm, uval_hbm), (ipos_hbm, ival_hbm)):
        pltpu.sync_copy(pos_hbm, pos_v)
        pltpu.sync_copy(val_hbm, val_v)

        def merge(t, carry):
            s = pl.ds(t * 16, 16)
            pos = pos_v[s]
            m = (pos >= base) & (pos < base + BPW)
            plsc.addupdate_scatter(acc_v, [pos - base], val_v[s], mask=m)
            return carry
        lax.fori_loop(0, NW * CAP // 16, merge, 0)

    def finish(g, carry):
        s = pl.ds(g * 16, 16)
        out_v[s] = 1.0 / (1.0 + jnp.exp(-acc_v[s]))
        return carry
    lax.fori_loop(0, BPW // 16, finish, 0)
    pltpu.sync_copy(out_v, out_hbm.at[pl.ds(base, BPW)])


@jax.jit
def _launch(u_idx, i_idx, utT, itT, wb):
    mesh = plsc.VectorSubcoreMesh(
        core_axis_name="c", subcore_axis_name="s",
        num_cores=NUM_CORES, num_subcores=NUM_SUBCORES)
    cparams = pltpu.CompilerParams(needs_layout_passes=False)
    lists = jax.ShapeDtypeStruct((NW * CAP,), jnp.int32)
    listsf = jax.ShapeDtypeStruct((NW * CAP,), jnp.float32)
    k1 = functools.partial(
        pl.kernel,
        out_type=(lists, listsf, lists, listsf),
        mesh=mesh,
        compiler_params=cparams,
        scratch_types=[
            pltpu.VMEM((BATCH,), jnp.int32),
            pltpu.VMEM((EMB_K, CW), jnp.float32),
            pltpu.VMEM((EMB_K, CW), jnp.float32),
            pltpu.VMEM((CAP + PAD,), jnp.int32),
            pltpu.VMEM((CAP + PAD,), jnp.int32),
            pltpu.VMEM((CAP + PAD,), jnp.int32),
            pltpu.VMEM((CAP + PAD,), jnp.int32),
            pltpu.VMEM((CAP + PAD,), jnp.int32),
            pltpu.VMEM((CAP + PAD,), jnp.int32),
            pltpu.VMEM((CAP + PAD,), jnp.int32),
            pltpu.VMEM((CAP + PAD,), jnp.float32),
            pltpu.VMEM((CAP + PAD,), jnp.int32),
            pltpu.VMEM((CAP + PAD,), jnp.float32),
            pltpu.VMEM((WLEN,), jnp.float32),
            pltpu.SemaphoreType.DMA,
            pltpu.SemaphoreType.DMA,
        ],
    )(_k1)
    upos, uval, ipos, ival = k1(u_idx, i_idx, utT, itT, wb)
    k2 = functools.partial(
        pl.kernel,
        out_type=jax.ShapeDtypeStruct((BATCH,), jnp.float32),
        mesh=mesh,
        compiler_params=cparams,
        scratch_types=[
            pltpu.VMEM((NW * CAP,), jnp.int32),
            pltpu.VMEM((NW * CAP,), jnp.float32),
            pltpu.VMEM((WLEN,), jnp.float32),
            pltpu.VMEM((BPW,), jnp.float32),
            pltpu.VMEM((BPW,), jnp.float32),
        ],
    )(_k2)
    return k2(upos, uval, ipos, ival, wb)


def kernel(x, user_table, item_table, W, b):
    u_idx = x[:, 0].astype(jnp.int32)
    i_idx = x[:, 1].astype(jnp.int32)
    wb = jnp.concatenate(
        [W.reshape(-1), b.reshape(-1),
         jnp.zeros((WLEN - 2 * EMB_K - 1,), jnp.float32)])
    return _launch(u_idx, i_idx, user_table.T, item_table.T, wb)
